# baseline (speedup 1.0000x reference)
import jax
import jax.numpy as jnp
from jax.experimental import pallas as pl
from jax.experimental.pallas import tpu as pltpu

B, T, C, NH, KS = 4, 2048, 768, 12, 16
HD = C // NH
G = NH * KS


def _dot(a, b, dims):
    return jax.lax.dot_general(a, b, (dims, ((), ())),
                               preferred_element_type=jnp.float32)


def _fused_kernel(x_ref, x16_ref, wqkv_ref, wp_ref, kms_ref, vm_ref, m_ref,
                  g_ref, gt_ref, o_ref):
    bf16 = jnp.bfloat16
    x16 = x16_ref[0]
    kT = _dot(wqkv_ref[C:2 * C, :], x16, ((1,), (1,)))
    v16 = _dot(x16, wqkv_ref[2 * C:, :], ((1,), (1,)))
    kbd = (jnp.concatenate([kT] * NH, axis=1) * kms_ref[:]).astype(bf16)
    vbd = (jnp.concatenate([v16] * NH, axis=0) * vm_ref[:]).astype(bf16)
    m_mat = _dot(wqkv_ref[:C, :].astype(bf16), kbd, ((0,), (0,))).astype(bf16)
    p_mat = _dot(vbd, wp_ref[:].astype(bf16), ((1,), (1,))).astype(bf16)

    xb = x_ref[0].astype(bf16)
    lg = _dot(xb, m_mat, ((1,), (0,)))
    e = jnp.exp(lg)
    e = jnp.concatenate([e[:KS] * m_ref[:], e[KS:]], axis=0)
    s = _dot(e, g_ref[:], ((1,), (0,)))
    r = 1.0 / jnp.maximum(s, 1e-30)
    rf = _dot(r, gt_ref[:], ((1,), (0,)))
    o_ref[0] = _dot((e * rf).astype(bf16), p_mat, ((1,), (0,)))


def kernel(x, Wqkv, Wproj, Wr):
    del Wr
    f32 = jnp.float32

    rows_c = jnp.arange(C)[:, None] // HD
    cols_g = jnp.arange(G)[None, :] // KS
    kms = jnp.where(rows_c == cols_g, f32(1.0 / (HD ** 0.5)), f32(0.0))
    vm = jnp.where(cols_g.T == rows_c.T, f32(1.0), f32(0.0))
    j_in_g = (jnp.arange(G) % KS)[None, :]
    j_ids = jnp.arange(KS)[:, None]
    mtab = jnp.where(j_in_g <= j_ids, f32(1.0), f32(0.0))
    h_ids = jnp.arange(16)[None, :]
    gmat = jnp.where(cols_g.T == h_ids, f32(1.0), f32(0.0))
    gtmat = gmat.T

    out = pl.pallas_call(
        _fused_kernel,
        grid=(B,),
        in_specs=[
            pl.BlockSpec((1, T, C), lambda b: (b, 0, 0)),
            pl.BlockSpec((1, KS, C), lambda b: (b, 0, 0)),
            pl.BlockSpec((3 * C, C), lambda b: (0, 0)),
            pl.BlockSpec((C, C), lambda b: (0, 0)),
            pl.BlockSpec((C, G), lambda b: (0, 0)),
            pl.BlockSpec((G, C), lambda b: (0, 0)),
            pl.BlockSpec((KS, G), lambda b: (0, 0)),
            pl.BlockSpec((G, 16), lambda b: (0, 0)),
            pl.BlockSpec((16, G), lambda b: (0, 0)),
        ],
        out_specs=pl.BlockSpec((1, T, C), lambda b: (b, 0, 0)),
        out_shape=jax.ShapeDtypeStruct((B, T, C), f32),
        compiler_params=pltpu.CompilerParams(
            dimension_semantics=("arbitrary",)),
    )(x, x, Wqkv, Wproj, kms, vm, mtab, gmat, gtmat)
    return out
